# baseline (speedup 1.0000x reference)
import functools
import math

import jax
import jax.numpy as jnp
from jax import lax
from jax.experimental import pallas as pl
from jax.experimental.pallas import tpu as pltpu
from jax.experimental.pallas import tpu_sc as plsc

NB = 16
C = 64
HW = 1024
NE = 1024
N_TOK = NB * HW
BPS = 4
NSTEP = NB // BPS
NW = 32
P = N_TOK // NW
HPB = HW // P


def _assign(x, emb, em2, e2):
    mm2 = lax.dot_general(em2, x, (((1,), (0,)), ((), ())),
                          preferred_element_type=jnp.float32)
    xn = jnp.sum(x * x, axis=0, keepdims=True)
    d2 = (xn + e2) + mm2
    mind = jnp.min(d2, axis=0, keepdims=True)
    eqm = d2 == mind
    iota = lax.broadcasted_iota(jnp.int32, (NE, HW), 0)
    idxv = jnp.min(jnp.where(eqm, iota, NE), axis=0, keepdims=True)
    onehot = (iota == idxv).astype(jnp.float32)
    h = jnp.sum(onehot, axis=1, keepdims=True)
    return idxv, h


def _tc_body(x_ref, e_ref, em2_ref, idx_ref, perp_ref, e2_ref, hist_ref):
    b = pl.program_id(0)
    emb = e_ref[...]

    @pl.when(b == 0)
    def _init():
        hist_ref[...] = jnp.zeros_like(hist_ref)
        e2_ref[...] = jnp.sum(emb * emb, axis=1, keepdims=True)

    em2 = em2_ref[...]
    e2 = e2_ref[...]
    for j in range(BPS):
        idxv, h = _assign(x_ref[j], emb, em2, e2)
        idx_ref[j] = idxv
        hist_ref[...] += h

    @pl.when(b == NSTEP - 1)
    def _fini():
        probs = hist_ref[...] * (1.0 / N_TOK)
        ent = -jnp.sum(probs * jnp.log(probs + 1e-10))
        perp_ref[0, 0] = jnp.exp(ent)


def _tc_call(x3, embed, embed_m2):
    return pl.pallas_call(
        _tc_body,
        grid=(NSTEP,),
        in_specs=[
            pl.BlockSpec((BPS, C, HW), lambda b: (b, 0, 0)),
            pl.BlockSpec((NE, C), lambda b: (0, 0)),
            pl.BlockSpec((NE, C), lambda b: (0, 0)),
        ],
        out_specs=[
            pl.BlockSpec((BPS, 1, HW), lambda b: (b, 0, 0)),
            pl.BlockSpec(memory_space=pltpu.SMEM),
        ],
        out_shape=[
            jax.ShapeDtypeStruct((NB, 1, HW), jnp.int32),
            jax.ShapeDtypeStruct((1, 1), jnp.float32),
        ],
        scratch_shapes=[
            pltpu.VMEM((NE, 1), jnp.float32),
            pltpu.VMEM((NE, 1), jnp.float32),
        ],
    )(x3, embed, embed_m2)


CPAD = 128
PH = P // 2


@functools.partial(
    pl.kernel,
    out_type=[
        jax.ShapeDtypeStruct((NB, C, HW), jnp.float32),
        jax.ShapeDtypeStruct((NW, 16), jnp.float32),
    ],
    mesh=plsc.VectorSubcoreMesh(core_axis_name="c", subcore_axis_name="s"),
    compiler_params=pltpu.CompilerParams(needs_layout_passes=False),
    scratch_types=[
        pltpu.VMEM((P,), jnp.int32),
        pltpu.VMEM((PH, CPAD), jnp.float32),
        pltpu.VMEM((C * P,), jnp.float32),
        pltpu.VMEM((C * P,), jnp.float32),
        pltpu.VMEM((16,), jnp.float32),
        pltpu.SemaphoreType.DMA,
    ],
)
def _sc_gather(x_hbm, ep_hbm, idx_hbm, zq_hbm, part_hbm,
               idx_v, rows_v, x_v, zq_v, part_v, sem):
    cid = lax.axis_index("c")
    sid = lax.axis_index("s")
    wid = sid * 2 + cid
    b = wid // HPB
    p0 = (wid % HPB) * P

    pltpu.sync_copy(idx_hbm.at[b, 0, pl.ds(p0, P)], idx_v)
    xcps = [pltpu.async_copy(x_hbm.at[b, c, pl.ds(p0, P)],
                             x_v.at[pl.ds(c * P, P)], sem)
            for c in range(C)]
    for cp in xcps:
        cp.wait()

    iota_p = lax.broadcasted_iota(jnp.int32, (16,), 0) * P
    zero = jnp.zeros((16,), jnp.float32)

    accs = (zero,) * (C // 16)
    for half in range(2):
        pltpu.async_copy(ep_hbm.at[idx_v.at[pl.ds(half * PH, PH)]],
                         rows_v, sem).wait()

        def body(r, accs, _half=half):
            accs = list(accs)
            for k in range(C // 16):
                lin = iota_p + (k * 16 * P + _half * PH + r)
                v = rows_v[r, pl.ds(k * 16, 16)]
                gx = plsc.load_gather(x_v, [lin])
                d = v - gx
                accs[k] = accs[k] + d * d
                plsc.store_scatter(zq_v, [lin], gx + d)
            return tuple(accs)

        accs = lax.fori_loop(0, PH, body, accs)

    acc = accs[0]
    for k in range(1, C // 16):
        acc = acc + accs[k]
    part_v[...] = acc
    zcps = [pltpu.async_copy(zq_v.at[pl.ds(c * P, P)],
                             zq_hbm.at[b, c, pl.ds(p0, P)], sem)
            for c in range(C)]
    for cp in zcps:
        cp.wait()
    pltpu.sync_copy(part_v, part_hbm.at[wid])


def kernel(inputs, embed):
    x3 = inputs.reshape(NB, C, HW)
    idx3, perp = _tc_call(x3, embed, embed * (-2.0))
    embed_pad = jnp.pad(embed, ((0, 0), (0, CPAD - C)))
    zq, parts = _sc_gather(x3, embed_pad, idx3)
    loss = 1.25 * jnp.sum(parts) / (NB * C * HW)
    z_q_out = zq.reshape(NB, C, 32, 32)
    kldiv_r = math.log(NE) * HW * jnp.ones((NB, 1), dtype=jnp.float32)
    return (z_q_out, loss, kldiv_r, perp[0, 0])


# speedup vs baseline: 2.8487x; 2.8487x over previous
import math

import jax
import jax.numpy as jnp
from jax import lax
from jax.experimental import pallas as pl
from jax.experimental.pallas import tpu as pltpu

NB = 16
C = 64
HW = 1024
NE = 1024
N_TOK = NB * HW
BPS = 8
NSTEP = NB // BPS


def _quantize(x, emb, em2, e2):
    mm2 = lax.dot_general(em2, x, (((1,), (0,)), ((), ())),
                          preferred_element_type=jnp.float32)
    xn = jnp.sum(x * x, axis=0, keepdims=True)
    d2 = (xn + e2) + mm2
    mind = jnp.min(d2, axis=0, keepdims=True)
    eqm = d2 == mind
    m = jnp.where(eqm, 1.0, 0.0)
    zq = lax.dot_general(emb, m, (((0,), (0,)), ((), ())),
                         preferred_element_type=jnp.float32)
    diff = zq - x
    zq_st = x + diff
    s = jnp.sum(diff * diff)
    h = jnp.sum(m, axis=1, keepdims=True)
    return zq_st, s, h, eqm


def _exact_onehot(eqm):
    iota = lax.broadcasted_iota(jnp.int32, eqm.shape, 0)
    idxv = jnp.min(jnp.where(eqm, iota, NE), axis=0, keepdims=True)
    return (iota == idxv).astype(jnp.float32)


def _vq_body(x_ref, e_ref, em2_ref, zq_ref, loss_ref, perp_ref,
             e2_ref, hist_ref, acc_ref):
    b = pl.program_id(0)
    emb = e_ref[...]

    @pl.when(b == 0)
    def _init():
        acc_ref[0] = 0.0
        hist_ref[...] = jnp.zeros_like(hist_ref)
        e2_ref[...] = jnp.sum(emb * emb, axis=1, keepdims=True)

    em2 = em2_ref[...]
    e2 = e2_ref[...]

    res = [_quantize(x_ref[j], emb, em2, e2) for j in range(BPS)]
    for j, (zq_st, _, _, _) in enumerate(res):
        zq_ref[j] = zq_st
    s_all = res[0][1]
    h_all = res[0][2]
    for j in range(1, BPS):
        s_all = s_all + res[j][1]
        h_all = h_all + res[j][2]
    acc_ref[0] += s_all
    hist_ref[...] += h_all

    @pl.when(jnp.sum(h_all) > BPS * HW + 0.5)
    def _fix():
        for j in range(BPS):
            _, s1, h1, eqm = res[j]
            x_j = x_ref[j]
            onehot = _exact_onehot(eqm)
            zq2 = lax.dot_general(emb, onehot, (((0,), (0,)), ((), ())),
                                  preferred_element_type=jnp.float32)
            diff2 = zq2 - x_j
            zq_ref[j] = x_j + diff2
            acc_ref[0] += jnp.sum(diff2 * diff2) - s1
            hist_ref[...] += jnp.sum(onehot, axis=1, keepdims=True) - h1

    @pl.when(b == NSTEP - 1)
    def _fini():
        loss_ref[0, 0] = 1.25 * acc_ref[0] / (NB * C * HW)
        probs = hist_ref[...] * (1.0 / N_TOK)
        ent = -jnp.sum(probs * jnp.log(probs + 1e-10))
        perp_ref[0, 0] = jnp.exp(ent)


def _vq_call(x3, embed, embed_m2, interpret=False):
    return pl.pallas_call(
        _vq_body,
        grid=(NSTEP,),
        in_specs=[
            pl.BlockSpec((BPS, C, HW), lambda b: (b, 0, 0)),
            pl.BlockSpec((NE, C), lambda b: (0, 0)),
            pl.BlockSpec((NE, C), lambda b: (0, 0)),
        ],
        out_specs=[
            pl.BlockSpec((BPS, C, HW), lambda b: (b, 0, 0)),
            pl.BlockSpec(memory_space=pltpu.SMEM),
            pl.BlockSpec(memory_space=pltpu.SMEM),
        ],
        out_shape=[
            jax.ShapeDtypeStruct((NB, C, HW), jnp.float32),
            jax.ShapeDtypeStruct((1, 1), jnp.float32),
            jax.ShapeDtypeStruct((1, 1), jnp.float32),
        ],
        scratch_shapes=[
            pltpu.VMEM((NE, 1), jnp.float32),
            pltpu.VMEM((NE, 1), jnp.float32),
            pltpu.SMEM((1,), jnp.float32),
        ],
        interpret=interpret,
    )(x3, embed, embed_m2)


def kernel(inputs, embed):
    x3 = inputs.reshape(NB, C, HW)
    zq, loss, perp = _vq_call(x3, embed, embed * (-2.0))
    z_q_out = zq.reshape(NB, C, 32, 32)
    kldiv_r = math.log(NE) * HW * jnp.ones((NB, 1), dtype=jnp.float32)
    return (z_q_out, loss[0, 0], kldiv_r, perp[0, 0])


# speedup vs baseline: 2.9267x; 1.0274x over previous
import math

import jax
import jax.numpy as jnp
from jax import lax
from jax.experimental import pallas as pl
from jax.experimental.pallas import tpu as pltpu

NB = 16
C = 64
HW = 1024
NE = 1024
N_TOK = NB * HW
BPS = 4
NSTEP = NB // BPS


def _quantize(x, emb, em2, e2):
    mm2 = lax.dot_general(em2, x, (((1,), (0,)), ((), ())),
                          preferred_element_type=jnp.float32)
    xn = jnp.sum(x * x, axis=0, keepdims=True)
    d2 = (xn + e2) + mm2
    mind = jnp.min(d2, axis=0, keepdims=True)
    eqm = d2 == mind
    m = jnp.where(eqm, 1.0, 0.0)
    zq = lax.dot_general(emb, m, (((0,), (0,)), ((), ())),
                         preferred_element_type=jnp.float32)
    diff = zq - x
    zq_st = x + diff
    s = jnp.sum(diff * diff)
    h = jnp.sum(m, axis=1, keepdims=True)
    return zq_st, s, h, eqm


def _exact_onehot(eqm):
    iota = lax.broadcasted_iota(jnp.int32, eqm.shape, 0)
    idxv = jnp.min(jnp.where(eqm, iota, NE), axis=0, keepdims=True)
    return (iota == idxv).astype(jnp.float32)


def _vq_body(x_ref, e_ref, zq_ref, loss_ref, perp_ref,
             em2_ref, e2_ref, hist_ref, acc_ref):
    b = pl.program_id(0)
    emb = e_ref[...]

    @pl.when(b == 0)
    def _init():
        acc_ref[0] = 0.0
        hist_ref[...] = jnp.zeros_like(hist_ref)
        e2_ref[...] = jnp.sum(emb * emb, axis=1, keepdims=True)
        em2_ref[...] = emb * (-2.0)

    em2 = em2_ref[...]
    e2 = e2_ref[...]

    res = [_quantize(x_ref[j], emb, em2, e2) for j in range(BPS)]
    for j, (zq_st, _, _, _) in enumerate(res):
        zq_ref[j] = zq_st
    s_all = res[0][1]
    h_all = res[0][2]
    for j in range(1, BPS):
        s_all = s_all + res[j][1]
        h_all = h_all + res[j][2]
    acc_ref[0] += s_all
    hist_ref[...] += h_all

    @pl.when(jnp.sum(h_all) > BPS * HW + 0.5)
    def _fix():
        for j in range(BPS):
            _, s1, h1, eqm = res[j]
            x_j = x_ref[j]
            onehot = _exact_onehot(eqm)
            zq2 = lax.dot_general(emb, onehot, (((0,), (0,)), ((), ())),
                                  preferred_element_type=jnp.float32)
            diff2 = zq2 - x_j
            zq_ref[j] = x_j + diff2
            acc_ref[0] += jnp.sum(diff2 * diff2) - s1
            hist_ref[...] += jnp.sum(onehot, axis=1, keepdims=True) - h1

    @pl.when(b == NSTEP - 1)
    def _fini():
        loss_ref[0, 0] = 1.25 * acc_ref[0] / (NB * C * HW)
        probs = hist_ref[...] * (1.0 / N_TOK)
        ent = -jnp.sum(probs * jnp.log(probs + 1e-10))
        perp_ref[0, 0] = jnp.exp(ent)


def _vq_call(x3, embed, interpret=False):
    return pl.pallas_call(
        _vq_body,
        grid=(NSTEP,),
        in_specs=[
            pl.BlockSpec((BPS, C, HW), lambda b: (b, 0, 0)),
            pl.BlockSpec((NE, C), lambda b: (0, 0)),
        ],
        out_specs=[
            pl.BlockSpec((BPS, C, HW), lambda b: (b, 0, 0)),
            pl.BlockSpec(memory_space=pltpu.SMEM),
            pl.BlockSpec(memory_space=pltpu.SMEM),
        ],
        out_shape=[
            jax.ShapeDtypeStruct((NB, C, HW), jnp.float32),
            jax.ShapeDtypeStruct((1, 1), jnp.float32),
            jax.ShapeDtypeStruct((1, 1), jnp.float32),
        ],
        scratch_shapes=[
            pltpu.VMEM((NE, C), jnp.float32),
            pltpu.VMEM((NE, 1), jnp.float32),
            pltpu.VMEM((NE, 1), jnp.float32),
            pltpu.SMEM((1,), jnp.float32),
        ],
        interpret=interpret,
    )(x3, embed)


def kernel(inputs, embed):
    x3 = inputs.reshape(NB, C, HW)
    zq, loss, perp = _vq_call(x3, embed)
    z_q_out = zq.reshape(NB, C, 32, 32)
    kldiv_r = math.log(NE) * HW * jnp.ones((NB, 1), dtype=jnp.float32)
    return (z_q_out, loss[0, 0], kldiv_r, perp[0, 0])
